# baseline (speedup 1.0000x reference)
import jax
import jax.numpy as jnp
from jax import lax
from jax.experimental import pallas as pl

_B, _C, _H, _W = 64, 768, 14, 14
_S = _H * _W
_E = 64


def _body(x_ref, w_ref, b_ref, o_ref):
    xb = x_ref[0]
    w = w_ref[...]
    m1 = lax.dot_general(
        w, xb, (((1,), (0,)), ((), ())), preferred_element_type=jnp.float32
    )
    logits = jnp.sum(m1, axis=1) * (1.0 / _S) + b_ref[0]
    mx = jnp.max(logits)
    e = jnp.exp(logits - mx)
    o_ref[0, 0, :] = e / jnp.sum(e)


def kernel(x, W, b):
    x2 = x.reshape(_B, _C, _S)
    b2 = b.reshape(1, _E)
    return pl.pallas_call(
        _body,
        grid=(_B,),
        in_specs=[
            pl.BlockSpec((1, _C, _S), lambda i: (i, 0, 0)),
            pl.BlockSpec((_E, _C), lambda i: (0, 0)),
            pl.BlockSpec((1, _E), lambda i: (0, 0)),
        ],
        out_specs=pl.BlockSpec((1, 1, _E), lambda i: (i, 0, 0)),
        out_shape=jax.ShapeDtypeStruct((_B, 1, _E), jnp.float32),
    )(x2, W, b2).reshape(_B, _E)
